# baseline (speedup 1.0000x reference)
import functools
import jax
import jax.numpy as jnp
from jax import lax
from jax.experimental import pallas as pl
from jax.experimental.pallas import tpu as pltpu
from jax.experimental.pallas import tpu_sc as plsc

_N = 10000
_C = 128
_ROWS = 50
_T = 16
_NBLK = _N // _T
_NW = 32
_NJ = (_NBLK + _NW - 1) // _NW


def _sc_body(x_hbm, o_hbm, buf, xv, isem, osem):
    wid = lax.axis_index("s") * 2 + lax.axis_index("c")

    z16 = jnp.zeros((16,), jnp.float32)

    def zbody(i, c):
        a = i // _ROWS
        r = i - a * _ROWS
        for k in range(8):
            buf[a, r, pl.ds(16 * k, 16)] = z16
        return c

    lax.fori_loop(0, _T * _ROWS, zbody, 0)

    def body(j, carry):
        blk = wid + _NW * j

        @pl.when(blk < _NBLK)
        def _():
            base = blk * _T
            g = pltpu.make_async_copy(x_hbm.at[pl.ds(base, _T), :], xv, isem)
            g.start()
            g.wait()
            for a in range(_T):
                for k in range(8):
                    buf[a, 0, pl.ds(16 * k, 16)] = xv[a, pl.ds(16 * k, 16)]
                    buf[a, 25, pl.ds(16 * k, 16)] = xv[
                        a, pl.ds(_C + 16 * k, 16)
                    ]
            s = pltpu.make_async_copy(
                buf, o_hbm.at[pl.ds(base, _T), :, :], osem
            )
            s.start()
            s.wait()

        return carry

    lax.fori_loop(0, _NJ, body, 0)


def kernel(atom_embeddings):
    mesh = plsc.VectorSubcoreMesh(core_axis_name="c", subcore_axis_name="s")
    f = functools.partial(
        pl.kernel,
        out_type=jax.ShapeDtypeStruct((_N, _ROWS, _C), jnp.float32),
        mesh=mesh,
        scratch_types=[
            pltpu.VMEM((_T, _ROWS, _C), jnp.float32),
            pltpu.VMEM((_T, 2 * _C), jnp.float32),
            pltpu.SemaphoreType.DMA,
            pltpu.SemaphoreType.DMA,
        ],
    )(_sc_body)
    return f(atom_embeddings)


# speedup vs baseline: 1.1414x; 1.1414x over previous
import jax
import jax.numpy as jnp
from jax.experimental import pallas as pl
from jax.experimental.pallas import tpu as pltpu

_N = 10000
_C = 128
_ROWS = 50
_A = 400
_NBLK = _N // _A
_NBUF = 3


def _body(x_hbm, o_hbm, *refs):
    bufs = refs[:_NBUF]
    xvs = refs[_NBUF:2 * _NBUF]
    isem, osem = refs[2 * _NBUF], refs[2 * _NBUF + 1]

    def in_copy(blk, b):
        return pltpu.make_async_copy(
            x_hbm.at[pl.ds(blk * _A, _A), :], xvs[b], isem.at[b]
        )

    def out_copy(blk, b):
        return pltpu.make_async_copy(
            bufs[b], o_hbm.at[pl.ds(blk * _A, _A), :, :], osem.at[b]
        )

    for b in range(_NBUF):
        in_copy(b, b).start()

    for i in range(_NBLK):
        b = i % _NBUF
        if i >= _NBUF:
            out_copy(i - _NBUF, b).wait()
        else:
            bufs[b][...] = jnp.zeros(bufs[b].shape, bufs[b].dtype)
        in_copy(i, b).wait()
        xb = xvs[b][...]
        bufs[b][:, 0:1, :] = xb[:, :_C].reshape(_A, 1, _C)
        bufs[b][:, 25:26, :] = xb[:, _C:].reshape(_A, 1, _C)
        out_copy(i, b).start(priority=i % 2)
        if i + _NBUF < _NBLK:
            in_copy(i + _NBUF, b).start()

    for i in range(_NBLK - _NBUF, _NBLK):
        out_copy(i, i % _NBUF).wait()


def kernel(atom_embeddings):
    return pl.pallas_call(
        _body,
        in_specs=[pl.BlockSpec(memory_space=pltpu.MemorySpace.HBM)],
        out_specs=pl.BlockSpec(memory_space=pltpu.MemorySpace.HBM),
        out_shape=jax.ShapeDtypeStruct((_N, _ROWS, _C), atom_embeddings.dtype),
        scratch_shapes=[pltpu.VMEM((_A, _ROWS, _C), jnp.float32)] * _NBUF
        + [pltpu.VMEM((_A, 2 * _C), jnp.float32)] * _NBUF
        + [
            pltpu.SemaphoreType.DMA((_NBUF,)),
            pltpu.SemaphoreType.DMA((_NBUF,)),
        ],
    )(atom_embeddings)
